# baseline (speedup 1.0000x reference)
import functools

import jax
import jax.numpy as jnp
from jax import lax
from jax.experimental import pallas as pl
from jax.experimental.pallas import tpu as pltpu
from jax.experimental.pallas import tpu_sc as plsc

STRIDES = (4.0, 8.0, 16.0, 32.0)
ANCHOR_SIZES = (4.0, 8.0, 16.0)
NMS_THR = 0.5
K_CAND = 256
K_KEEP = 32
OUT = 14
A = 3
C = 256
B = 2
LVL_HW = ((128, 128), (64, 64), (32, 32), (16, 16))
N_RAW = sum(A * h * w for h, w in LVL_HW)
N_PAD = 65536
NPROB = B * 4
NPTS = OUT * OUT
TOT_ROWS = NPROB * K_KEEP * NPTS

_LVL_OFF = []
_off = 0
for _h, _w in LVL_HW:
    _LVL_OFF.append(_off)
    _off += B * _h * _w
TABLE_ROWS = _off


def _decode_body(stride, H, W, Ht, feat_ref, wall_ref, msk_ref,
                 x1_ref, y1_ref, x2_ref, y2_ref):
    f2 = feat_ref[0].reshape(C, Ht * W)
    mm = jnp.dot(wall_ref[...], f2, preferred_element_type=jnp.float32)
    mm = mm.reshape(16, Ht, W)
    ybase = pl.program_id(1) * Ht
    xs = lax.broadcasted_iota(jnp.int32, (Ht, W), 1).astype(jnp.float32) + 0.5
    ys = (lax.broadcasted_iota(jnp.int32, (Ht, W), 0) + ybase
          ).astype(jnp.float32) + 0.5
    msk_l = [[] for _ in range(4)]
    x1_l, y1_l, x2_l, y2_l = [], [], [], []
    for a in range(A):
        obj = mm[a]
        dx = mm[A + 4 * a + 0]
        dy = mm[A + 4 * a + 1]
        dw = mm[A + 4 * a + 2]
        dh = mm[A + 4 * a + 3]
        sz = ANCHOR_SIZES[a]
        ncx = xs + jnp.tanh(dx) * sz
        ncy = ys + jnp.tanh(dy) * sz
        nw = sz * jnp.exp(jnp.clip(dw, -2.0, 2.0))
        nh = sz * jnp.exp(jnp.clip(dh, -2.0, 2.0))
        bcx = ncx * stride
        bcy = ncy * stride
        bw = nw * stride
        bh = nh * stride
        lvl = jnp.clip(jnp.floor(3.0 + jnp.log2(
            jnp.sqrt(jnp.maximum(bw * bh, 1e-6)) / 224.0)), 1.0, 4.0)
        for li in range(4):
            msk_l[li].append(jnp.where(lvl == float(li + 1), obj, -jnp.inf))
        x1_l.append(bcx - bw / 2)
        y1_l.append(bcy - bh / 2)
        x2_l.append(bcx + bw / 2)
        y2_l.append(bcy + bh / 2)
    msk_ref[0] = jnp.stack([jnp.stack(m, 0) for m in msk_l], 0)
    x1_ref[0] = jnp.stack(x1_l, 0)
    y1_ref[0] = jnp.stack(y1_l, 0)
    x2_ref[0] = jnp.stack(x2_l, 0)
    y2_ref[0] = jnp.stack(y2_l, 0)


def _decode_level(feat, wall, stride, H, W, Ht, interpret=False):
    grid = (B, H // Ht)
    plane = jax.ShapeDtypeStruct((B, A, H, W), jnp.float32)
    out_shapes = (jax.ShapeDtypeStruct((B, 4, A, H, W), jnp.float32),
                  plane, plane, plane, plane)
    plane_spec = pl.BlockSpec((1, A, Ht, W), lambda b, t: (b, 0, t, 0))
    return pl.pallas_call(
        functools.partial(_decode_body, stride, H, W, Ht),
        grid=grid,
        in_specs=[pl.BlockSpec((1, C, Ht, W), lambda b, t: (b, 0, t, 0)),
                  pl.BlockSpec((16, C), lambda b, t: (0, 0))],
        out_specs=(pl.BlockSpec((1, 4, A, Ht, W), lambda b, t: (b, 0, 0, t, 0)),
                   plane_spec, plane_spec, plane_spec, plane_spec),
        out_shape=out_shapes,
        interpret=interpret,
    )(feat, wall)


def _partner(arr, j, s_full):
    up = jnp.concatenate([arr[j:], arr[:j]], axis=0)
    dn = jnp.concatenate([arr[-j:], arr[:-j]], axis=0)
    return jnp.where(s_full, dn, up)


def _ce_stage(key, payloads, j, desc_f, riota):
    s_i = (riota // j) % 2
    sf = s_i.astype(jnp.float32) + jnp.zeros_like(key)
    s_full = sf > 0.5
    kp = _partner(key, j, s_full)
    gt_f = jnp.where(kp > key, 1.0, 0.0)
    lt_f = jnp.where(kp < key, 1.0, 0.0)
    m_f = desc_f * sf + (1.0 - desc_f) * (1.0 - sf)
    take = (m_f * lt_f + (1.0 - m_f) * gt_f) > 0.5
    new_key = jnp.where(take, kp, key)
    new_pl = [jnp.where(take, _partner(p, j, s_full), p) for p in payloads]
    return new_key, new_pl


def _sortmerge_body(sc_ref, x1_ref, y1_ref, x2_ref, y2_ref, *out_refs):
    key = sc_ref[...]
    payloads = [x1_ref[...], y1_ref[...], x2_ref[...], y2_ref[...]]

    riota2 = lax.broadcasted_iota(jnp.int32, (K_CAND, 1), 0)
    col_f = jnp.where(lax.broadcasted_iota(jnp.int32, (1, K_CAND), 1)
                      < (K_CAND // 2), 1.0, 0.0)
    k = 2
    while k <= K_CAND:
        j = k // 2
        while j >= 1:
            blk_f = (((riota2 // k) % 2)).astype(jnp.float32)
            desc_f = ((1.0 - blk_f) * col_f + blk_f * (1.0 - col_f))
            key, payloads = _ce_stage(key, payloads, j, desc_f, riota2)
            j //= 2
        k *= 2

    h = K_CAND // 2
    while h >= 1:
        ka = key[:, :h]
        kb = key[:, h:2 * h]
        take_b = kb > ka
        key = jnp.where(take_b, kb, ka)
        payloads = [jnp.where(take_b, p[:, h:2 * h], p[:, :h])
                    for p in payloads]
        if h > 1:
            col_desc_f = jnp.where(
                lax.broadcasted_iota(jnp.int32, (1, h), 1) < (h // 2),
                1.0, 0.0)
        else:
            col_desc_f = jnp.ones((1, 1), jnp.float32)
        j = K_CAND // 2
        while j >= 1:
            key, payloads = _ce_stage(key, payloads, j, col_desc_f, riota2)
            j //= 2
        h //= 2

    for a, arr in enumerate([key] + payloads):
        out_refs[a][...] = jnp.broadcast_to(arr, (K_CAND, 128))


def _sortmerge(scg, x1g, y1g, x2g, y2g, interpret=False):
    col = jax.ShapeDtypeStruct((K_CAND, NPROB * 128), jnp.float32)
    in_spec = pl.BlockSpec((K_CAND, K_CAND), lambda p: (0, p))
    col_spec = pl.BlockSpec((K_CAND, 128), lambda p: (0, p))
    outs = pl.pallas_call(
        _sortmerge_body,
        grid=(NPROB,),
        in_specs=[in_spec] * 5,
        out_specs=tuple([col_spec] * 5),
        out_shape=tuple([col] * 5),
        interpret=interpret,
    )(scg, x1g, y1g, x2g, y2g)
    return tuple(o.reshape(K_CAND, NPROB, 128)[:, :, 0] for o in outs)


def _nms_body(tv_ref, bx1_ref, by1_ref, bx2_ref, by2_ref,
              tvc_ref, bx1c_ref, by1c_ref, bx2c_ref, by2c_ref,
              sbx1_ref, sby1_ref, sbx2_ref, sby2_ref,
              i00_ref, i01_ref, i10_ref, i11_ref,
              w00_ref, w01_ref, w10_ref, w11_ref,
              supf_ref, keep_ref):
    tv = tv_ref[...]
    bx1 = bx1_ref[...]
    by1 = by1_ref[...]
    bx2 = bx2_ref[...]
    by2 = by2_ref[...]
    bx1_c = bx1c_ref[...]
    by1_c = by1c_ref[...]
    bx2_c = bx2c_ref[...]
    by2_c = by2c_ref[...]
    valid = tv > -1e30

    area = jnp.maximum(bx2 - bx1, 0.0) * jnp.maximum(by2 - by1, 0.0)
    area_c = (jnp.maximum(bx2_c - bx1_c, 0.0)
              * jnp.maximum(by2_c - by1_c, 0.0))
    ix1 = jnp.maximum(bx1_c[:, :, None], bx1[None, :, :])
    iy1 = jnp.maximum(by1_c[:, :, None], by1[None, :, :])
    ix2 = jnp.minimum(bx2_c[:, :, None], bx2[None, :, :])
    iy2 = jnp.minimum(by2_c[:, :, None], by2[None, :, :])
    inter = jnp.maximum(ix2 - ix1, 0.0) * jnp.maximum(iy2 - iy1, 0.0)
    iou = inter / (area_c[:, :, None] + area[None, :, :] - inter + 1e-9)
    supf_ref[...] = jnp.where(iou > NMS_THR, 1.0, 0.0)
    keep_ref[...] = jnp.where(valid, 1.0, 0.0)

    lane = lax.broadcasted_iota(jnp.int32, (NPROB, K_CAND), 1)

    def nms_body(i, carry):
        keep = keep_ref[...]
        ki = jnp.sum(keep * jnp.where(lane == i, 1.0, 0.0),
                     axis=1, keepdims=True)
        row = supf_ref[pl.ds(i, 1)].reshape(NPROB, K_CAND)
        sup = ki * row * jnp.where(lane > i, 1.0, 0.0)
        keep_ref[...] = keep * (1.0 - sup)
        return carry

    lax.fori_loop(0, K_CAND, nms_body, 0)

    keepf = keep_ref[...]
    keep = keepf > 0.5
    tri = jnp.where(lax.broadcasted_iota(jnp.int32, (K_CAND, K_CAND), 0)
                    <= lax.broadcasted_iota(jnp.int32, (K_CAND, K_CAND), 1),
                    1.0, 0.0)
    kc = jnp.dot(keepf, tri, preferred_element_type=jnp.float32)
    nc = jnp.dot(1.0 - keepf, tri, preferred_element_type=jnp.float32)
    nk_tot = kc[:, K_CAND - 1:K_CAND]
    rank = jnp.where(keep, kc - 1.0, nk_tot + nc - 1.0)

    srange = lax.broadcasted_iota(jnp.int32, (1, K_KEEP, 1), 1
                                  ).astype(jnp.float32)
    M = jnp.where(rank[:, None, :] == srange, 1.0, 0.0)
    v = jnp.sum(M * keepf[:, None, :], axis=2)
    sx1 = jnp.sum(M * bx1[:, None, :], axis=2) * v
    sy1 = jnp.sum(M * by1[:, None, :], axis=2) * v
    sx2 = jnp.sum(M * bx2[:, None, :], axis=2) * v
    sy2 = jnp.sum(M * by2[:, None, :], axis=2) * v
    sbx1_ref[...] = sx1
    sby1_ref[...] = sy1
    sbx2_ref[...] = sx2
    sby2_ref[...] = sy2

    pio = lax.broadcasted_iota(jnp.int32, (NPROB, 1), 0)
    li_c = pio % 4
    b_c = pio // 4
    stride_c = (1 << (li_c + 2)).astype(jnp.float32)
    w_c = lax.shift_right_logical(128, li_c)[:, :, None]
    h_c = w_c
    hw_c = lax.shift_right_logical(16384, 2 * li_c)
    off_c = jnp.where(li_c == 0, 0,
                      jnp.where(li_c == 1, _LVL_OFF[1],
                                jnp.where(li_c == 2, _LVL_OFF[2], _LVL_OFF[3])))
    base_c = (off_c + b_c * hw_c)[:, :, None, None]

    x1s = sx1 / stride_c
    y1s = sy1 / stride_c
    x2s = sx2 / stride_c
    y2s = sy2 / stride_c
    bw = jnp.maximum(x2s - x1s, 1e-3)
    bh = jnp.maximum(y2s - y1s, 1e-3)
    kio = lax.broadcasted_iota(jnp.int32, (1, 1, OUT), 2
                               ).astype(jnp.float32) + 0.5
    xs = x1s[:, :, None] + kio * bw[:, :, None] / OUT - 0.5
    ys = y1s[:, :, None] + kio * bh[:, :, None] / OUT - 0.5
    x0f = jnp.floor(xs)
    y0f = jnp.floor(ys)
    wx = xs - x0f
    wy = ys - y0f
    x0i = jnp.clip(x0f.astype(jnp.int32), 0, w_c - 1)
    x1i = jnp.clip(x0i + 1, 0, w_c - 1)
    y0i = jnp.clip(y0f.astype(jnp.int32), 0, h_c - 1)
    y1i = jnp.clip(y0i + 1, 0, h_c - 1)

    y0w = y0i * w_c
    y1w = y1i * w_c
    i00_ref[...] = base_c + y0w[:, :, :, None] + x0i[:, :, None, :]
    i01_ref[...] = base_c + y0w[:, :, :, None] + x1i[:, :, None, :]
    i10_ref[...] = base_c + y1w[:, :, :, None] + x0i[:, :, None, :]
    i11_ref[...] = base_c + y1w[:, :, :, None] + x1i[:, :, None, :]
    vw = v[:, :, None, None]
    w00_ref[...] = (1.0 - wy)[:, :, :, None] * (1.0 - wx)[:, :, None, :] * vw
    w01_ref[...] = (1.0 - wy)[:, :, :, None] * wx[:, :, None, :] * vw
    w10_ref[...] = wy[:, :, :, None] * (1.0 - wx)[:, :, None, :] * vw
    w11_ref[...] = wy[:, :, :, None] * wx[:, :, None, :] * vw


def _nms_select(sorted_cols, interpret=False):
    rows = [jnp.transpose(a) for a in sorted_cols]
    sb = jax.ShapeDtypeStruct((NPROB, K_KEEP), jnp.float32)
    idx = jax.ShapeDtypeStruct((NPROB, K_KEEP, OUT, OUT), jnp.int32)
    wgt = jax.ShapeDtypeStruct((NPROB, K_KEEP, OUT, OUT), jnp.float32)
    return pl.pallas_call(
        _nms_body,
        out_shape=(sb, sb, sb, sb, idx, idx, idx, idx, wgt, wgt, wgt, wgt),
        scratch_shapes=[pltpu.VMEM((K_CAND, NPROB, K_CAND), jnp.float32),
                        pltpu.VMEM((NPROB, K_CAND), jnp.float32)],
        interpret=interpret,
    )(*rows, *sorted_cols)


_SC_TILES = 32
TOT4 = 4 * TOT_ROWS
_TILE_ROWS = TOT4 // _SC_TILES
_SC_CHUNK = 112
_SC_STEPS = _TILE_ROWS // _SC_CHUNK


def _roi_sc_body(table, idx, out, ivall, rb0, rb1, gs0, gs1, ws0, ws1):
    wid = lax.axis_index("s") * 2 + lax.axis_index("c")
    base = wid * _TILE_ROWS

    pltpu.sync_copy(idx.at[pl.ds(base, _TILE_ROWS)], ivall)

    bufs = (rb0, rb1)
    gsems = (gs0, gs1)
    wsems = (ws0, ws1)
    pending_wb = [None, None]

    g_prev = pltpu.async_copy(
        table.at[ivall.at[pl.ds(0, _SC_CHUNK)]], rb0, gs0)
    for i in range(1, _SC_STEPS + 1):
        cur = (i - 1) % 2
        nxt = i % 2
        if i < _SC_STEPS:
            if pending_wb[nxt] is not None:
                pending_wb[nxt].wait()
            g_next = pltpu.async_copy(
                table.at[ivall.at[pl.ds(i * _SC_CHUNK, _SC_CHUNK)]],
                bufs[nxt], gsems[nxt])
        g_prev.wait()
        pending_wb[cur] = pltpu.async_copy(
            bufs[cur], out.at[pl.ds(base + (i - 1) * _SC_CHUNK, _SC_CHUNK)],
            wsems[cur])
        if i < _SC_STEPS:
            g_prev = g_next
    pending_wb[0].wait()
    pending_wb[1].wait()


def _roi_sc(table, idx):
    mesh = plsc.VectorSubcoreMesh(core_axis_name="c", subcore_axis_name="s",
                                  num_cores=2, num_subcores=16)
    fn = pl.kernel(
        _roi_sc_body,
        out_type=jax.ShapeDtypeStruct((TOT4, C // 2), jnp.int32),
        mesh=mesh,
        scratch_types=[
            pltpu.VMEM((_TILE_ROWS,), jnp.int32),
            pltpu.VMEM((_SC_CHUNK, C // 2), jnp.int32),
            pltpu.VMEM((_SC_CHUNK, C // 2), jnp.int32),
            pltpu.SemaphoreType.DMA,
            pltpu.SemaphoreType.DMA,
            pltpu.SemaphoreType.DMA,
            pltpu.SemaphoreType.DMA,
        ],
    )
    return fn(table, idx)


_CMB_ROWS = 1024


def _combine_body(g0_ref, g1_ref, g2_ref, g3_ref,
                  w0_ref, w1_ref, w2_ref, w3_ref, out_ref):
    out_ref[...] = (w0_ref[...] * g0_ref[...].astype(jnp.float32)
                    + w1_ref[...] * g1_ref[...].astype(jnp.float32)
                    + w2_ref[...] * g2_ref[...].astype(jnp.float32)
                    + w3_ref[...] * g3_ref[...].astype(jnp.float32))


def _combine(G, ws, interpret=False):
    nblk = TOT_ROWS // _CMB_ROWS
    g_spec = pl.BlockSpec((_CMB_ROWS, C), lambda t: (t, 0))
    gk_specs = [pl.BlockSpec((_CMB_ROWS, C),
                             functools.partial(lambda k, t: (k * nblk + t, 0),
                                               k))
                for k in range(4)]
    w_spec = pl.BlockSpec((_CMB_ROWS, 1), lambda t: (t, 0))
    return pl.pallas_call(
        _combine_body,
        grid=(nblk,),
        in_specs=gk_specs + [w_spec] * 4,
        out_specs=g_spec,
        out_shape=jax.ShapeDtypeStruct((TOT_ROWS, C), jnp.float32),
        interpret=interpret,
    )(G, G, G, G, *ws)


def _grid_layout(planes_per_level, broadcast_levels):
    flat = jnp.concatenate(planes_per_level, axis=-1)
    if broadcast_levels:
        flat = jnp.broadcast_to(flat[:, None, :], (B, 4, N_RAW))
        pad_val = 0.0
    else:
        pad_val = -jnp.inf
    flat = jnp.pad(flat, ((0, 0), (0, 0), (0, N_PAD - N_RAW)),
                   constant_values=pad_val)
    g = flat.reshape(B, 4, K_CAND, K_CAND)
    return g.transpose(2, 0, 1, 3).reshape(K_CAND, NPROB * K_CAND)


def kernel(p32, p16, p8, p4, W_obj_p4, W_reg_p4, W_obj_p8, W_reg_p8,
           W_obj_p16, W_reg_p16, W_obj_p32, W_reg_p32, _interpret=False):
    feats = (p4, p8, p16, p32)
    wobjs = (W_obj_p4, W_obj_p8, W_obj_p16, W_obj_p32)
    wregs = (W_reg_p4, W_reg_p8, W_reg_p16, W_reg_p32)
    tiles = (32, 64, 32, 16)

    msk_list, x1_list, y1_list, x2_list, y2_list = [], [], [], [], []
    for li in range(4):
        H, W = LVL_HW[li]
        wall = jnp.concatenate(
            [wobjs[li], wregs[li].reshape(4 * A, C),
             jnp.zeros((16 - A - 4 * A, C), jnp.float32)], axis=0)
        msk, x1p, y1p, x2p, y2p = _decode_level(
            feats[li], wall, STRIDES[li], H, W, tiles[li],
            interpret=_interpret)
        msk_list.append(msk.reshape(B, 4, A * H * W))
        x1_list.append(x1p.reshape(B, A * H * W))
        y1_list.append(y1p.reshape(B, A * H * W))
        x2_list.append(x2p.reshape(B, A * H * W))
        y2_list.append(y2p.reshape(B, A * H * W))

    scg = _grid_layout([m for m in msk_list], broadcast_levels=False)
    x1g = _grid_layout(x1_list, broadcast_levels=True)
    y1g = _grid_layout(y1_list, broadcast_levels=True)
    x2g = _grid_layout(x2_list, broadcast_levels=True)
    y2g = _grid_layout(y2_list, broadcast_levels=True)

    sorted_cols = _sortmerge(scg, x1g, y1g, x2g, y2g, interpret=_interpret)
    (sx1, sy1, sx2, sy2, i00, i01, i10, i11, w00, w01, w10, w11) = _nms_select(
        sorted_cols, interpret=_interpret)

    anchors = jnp.stack([sx1, sy1, sx2, sy2], axis=-1)
    anchors = anchors.reshape(B, 4 * K_KEEP, 4)

    table = jnp.concatenate(
        [jnp.transpose(f, (0, 2, 3, 1)).reshape(-1, C) for f in feats],
        axis=0).astype(jnp.bfloat16)

    idx_all = jnp.concatenate(
        [i.reshape(TOT_ROWS) for i in (i00, i01, i10, i11)])
    wflat = [w.reshape(TOT_ROWS, 1) for w in (w00, w01, w10, w11)]

    if _interpret:
        G = table[idx_all]
    else:
        table_i32 = lax.bitcast_convert_type(
            table.reshape(TABLE_ROWS, C // 2, 2), jnp.int32)
        G_i32 = _roi_sc(table_i32, idx_all)
        G = lax.bitcast_convert_type(G_i32, jnp.bfloat16).reshape(TOT4, C)
    ext_rows = _combine(G, wflat, interpret=_interpret)

    exts = ext_rows.reshape(B, 4, K_KEEP, OUT, OUT, C)
    exts = exts.transpose(0, 1, 2, 5, 3, 4).reshape(
        B, 4 * K_KEEP, C, OUT, OUT)
    return exts, anchors


# speedup vs baseline: 1.8760x; 1.8760x over previous
import functools

import jax
import jax.numpy as jnp
from jax import lax
from jax.experimental import pallas as pl
from jax.experimental.pallas import tpu as pltpu
from jax.experimental.pallas import tpu_sc as plsc

STRIDES = (4.0, 8.0, 16.0, 32.0)
ANCHOR_SIZES = (4.0, 8.0, 16.0)
NMS_THR = 0.5
K_CAND = 256
K_KEEP = 32
OUT = 14
A = 3
C = 256
B = 2
LVL_HW = ((128, 128), (64, 64), (32, 32), (16, 16))
N_RAW = sum(A * h * w for h, w in LVL_HW)
N_PAD = 65536
NPROB = B * 4
NPTS = OUT * OUT
TOT_ROWS = NPROB * K_KEEP * NPTS

_LVL_OFF = []
_off = 0
for _h, _w in LVL_HW:
    _LVL_OFF.append(_off)
    _off += B * _h * _w
TABLE_ROWS = _off


def _decode_body(stride, H, W, Ht, feat_ref, wall_ref, msk_ref,
                 x1_ref, y1_ref, x2_ref, y2_ref):
    f2 = feat_ref[0].reshape(C, Ht * W)
    mm = jnp.dot(wall_ref[...], f2, preferred_element_type=jnp.float32)
    mm = mm.reshape(16, Ht, W)
    ybase = pl.program_id(1) * Ht
    xs = lax.broadcasted_iota(jnp.int32, (Ht, W), 1).astype(jnp.float32) + 0.5
    ys = (lax.broadcasted_iota(jnp.int32, (Ht, W), 0) + ybase
          ).astype(jnp.float32) + 0.5
    msk_l = [[] for _ in range(4)]
    x1_l, y1_l, x2_l, y2_l = [], [], [], []
    for a in range(A):
        obj = mm[a]
        dx = mm[A + 4 * a + 0]
        dy = mm[A + 4 * a + 1]
        dw = mm[A + 4 * a + 2]
        dh = mm[A + 4 * a + 3]
        sz = ANCHOR_SIZES[a]
        ncx = xs + jnp.tanh(dx) * sz
        ncy = ys + jnp.tanh(dy) * sz
        nw = sz * jnp.exp(jnp.clip(dw, -2.0, 2.0))
        nh = sz * jnp.exp(jnp.clip(dh, -2.0, 2.0))
        bcx = ncx * stride
        bcy = ncy * stride
        bw = nw * stride
        bh = nh * stride
        lvl = jnp.clip(jnp.floor(3.0 + jnp.log2(
            jnp.sqrt(jnp.maximum(bw * bh, 1e-6)) / 224.0)), 1.0, 4.0)
        for li in range(4):
            msk_l[li].append(jnp.where(lvl == float(li + 1), obj, -jnp.inf))
        x1_l.append(bcx - bw / 2)
        y1_l.append(bcy - bh / 2)
        x2_l.append(bcx + bw / 2)
        y2_l.append(bcy + bh / 2)
    msk_ref[0] = jnp.stack([jnp.stack(m, 0) for m in msk_l], 0)
    x1_ref[0] = jnp.stack(x1_l, 0)
    y1_ref[0] = jnp.stack(y1_l, 0)
    x2_ref[0] = jnp.stack(x2_l, 0)
    y2_ref[0] = jnp.stack(y2_l, 0)


def _decode_level(feat, wall, stride, H, W, Ht, interpret=False):
    grid = (B, H // Ht)
    plane = jax.ShapeDtypeStruct((B, A, H, W), jnp.float32)
    out_shapes = (jax.ShapeDtypeStruct((B, 4, A, H, W), jnp.float32),
                  plane, plane, plane, plane)
    plane_spec = pl.BlockSpec((1, A, Ht, W), lambda b, t: (b, 0, t, 0))
    return pl.pallas_call(
        functools.partial(_decode_body, stride, H, W, Ht),
        grid=grid,
        in_specs=[pl.BlockSpec((1, C, Ht, W), lambda b, t: (b, 0, t, 0)),
                  pl.BlockSpec((16, C), lambda b, t: (0, 0))],
        out_specs=(pl.BlockSpec((1, 4, A, Ht, W), lambda b, t: (b, 0, 0, t, 0)),
                   plane_spec, plane_spec, plane_spec, plane_spec),
        out_shape=out_shapes,
        interpret=interpret,
    )(feat, wall)


def _partner(arr, j, s_full):
    up = jnp.concatenate([arr[j:], arr[:j]], axis=0)
    dn = jnp.concatenate([arr[-j:], arr[:-j]], axis=0)
    return jnp.where(s_full, dn, up)


def _ce_stage(key, payloads, j, desc_f, riota):
    s_i = (riota // j) % 2
    sf = s_i.astype(jnp.float32) + jnp.zeros_like(key)
    s_full = sf > 0.5
    kp = _partner(key, j, s_full)
    gt_f = jnp.where(kp > key, 1.0, 0.0)
    lt_f = jnp.where(kp < key, 1.0, 0.0)
    m_f = desc_f * sf + (1.0 - desc_f) * (1.0 - sf)
    take = (m_f * lt_f + (1.0 - m_f) * gt_f) > 0.5
    new_key = jnp.where(take, kp, key)
    new_pl = [jnp.where(take, _partner(p, j, s_full), p) for p in payloads]
    return new_key, new_pl


def _sortmerge_body(sc_ref, x1_ref, y1_ref, x2_ref, y2_ref, *out_refs):
    key = sc_ref[...]
    payloads = [x1_ref[...], y1_ref[...], x2_ref[...], y2_ref[...]]

    riota2 = lax.broadcasted_iota(jnp.int32, (K_CAND, 1), 0)
    col_f = jnp.where(lax.broadcasted_iota(jnp.int32, (1, K_CAND), 1)
                      < (K_CAND // 2), 1.0, 0.0)
    k = 2
    while k <= K_CAND:
        j = k // 2
        while j >= 1:
            blk_f = (((riota2 // k) % 2)).astype(jnp.float32)
            desc_f = ((1.0 - blk_f) * col_f + blk_f * (1.0 - col_f))
            key, payloads = _ce_stage(key, payloads, j, desc_f, riota2)
            j //= 2
        k *= 2

    h = K_CAND // 2
    while h >= 1:
        ka = key[:, :h]
        kb = key[:, h:2 * h]
        take_b = kb > ka
        key = jnp.where(take_b, kb, ka)
        payloads = [jnp.where(take_b, p[:, h:2 * h], p[:, :h])
                    for p in payloads]
        if h > 1:
            col_desc_f = jnp.where(
                lax.broadcasted_iota(jnp.int32, (1, h), 1) < (h // 2),
                1.0, 0.0)
        else:
            col_desc_f = jnp.ones((1, 1), jnp.float32)
        j = K_CAND // 2
        while j >= 1:
            key, payloads = _ce_stage(key, payloads, j, col_desc_f, riota2)
            j //= 2
        h //= 2

    for a, arr in enumerate([key] + payloads):
        out_refs[a][...] = jnp.broadcast_to(arr, (K_CAND, 128))


def _sortmerge(scg, x1g, y1g, x2g, y2g, interpret=False):
    col = jax.ShapeDtypeStruct((K_CAND, NPROB * 128), jnp.float32)
    in_spec = pl.BlockSpec((K_CAND, K_CAND), lambda p: (0, p))
    col_spec = pl.BlockSpec((K_CAND, 128), lambda p: (0, p))
    outs = pl.pallas_call(
        _sortmerge_body,
        grid=(NPROB,),
        in_specs=[in_spec] * 5,
        out_specs=tuple([col_spec] * 5),
        out_shape=tuple([col] * 5),
        interpret=interpret,
    )(scg, x1g, y1g, x2g, y2g)
    return tuple(o.reshape(K_CAND, NPROB, 128)[:, :, 0] for o in outs)


def _nms_body(tv_ref, bx1_ref, by1_ref, bx2_ref, by2_ref,
              tvc_ref, bx1c_ref, by1c_ref, bx2c_ref, by2c_ref,
              sbx1_ref, sby1_ref, sbx2_ref, sby2_ref,
              i00_ref, i01_ref, i10_ref, i11_ref,
              w00_ref, w01_ref, w10_ref, w11_ref,
              supf_ref, keep_ref):
    tv = tv_ref[...]
    bx1 = bx1_ref[...]
    by1 = by1_ref[...]
    bx2 = bx2_ref[...]
    by2 = by2_ref[...]
    bx1_c = bx1c_ref[...]
    by1_c = by1c_ref[...]
    bx2_c = bx2c_ref[...]
    by2_c = by2c_ref[...]
    valid = tv > -1e30

    area = jnp.maximum(bx2 - bx1, 0.0) * jnp.maximum(by2 - by1, 0.0)
    area_c = (jnp.maximum(bx2_c - bx1_c, 0.0)
              * jnp.maximum(by2_c - by1_c, 0.0))
    ix1 = jnp.maximum(bx1_c[:, :, None], bx1[None, :, :])
    iy1 = jnp.maximum(by1_c[:, :, None], by1[None, :, :])
    ix2 = jnp.minimum(bx2_c[:, :, None], bx2[None, :, :])
    iy2 = jnp.minimum(by2_c[:, :, None], by2[None, :, :])
    inter = jnp.maximum(ix2 - ix1, 0.0) * jnp.maximum(iy2 - iy1, 0.0)
    iou = inter / (area_c[:, :, None] + area[None, :, :] - inter + 1e-9)
    supf_ref[...] = jnp.where(iou > NMS_THR, 1.0, 0.0)
    keep_ref[...] = jnp.where(valid, 1.0, 0.0)

    lane = lax.broadcasted_iota(jnp.int32, (NPROB, K_CAND), 1)

    def nms_body(i, carry):
        keep = keep_ref[...]
        ki = jnp.sum(keep * jnp.where(lane == i, 1.0, 0.0),
                     axis=1, keepdims=True)
        row = supf_ref[pl.ds(i, 1)].reshape(NPROB, K_CAND)
        sup = ki * row * jnp.where(lane > i, 1.0, 0.0)
        keep_ref[...] = keep * (1.0 - sup)
        return carry

    lax.fori_loop(0, K_CAND, nms_body, 0)

    keepf = keep_ref[...]
    keep = keepf > 0.5
    tri = jnp.where(lax.broadcasted_iota(jnp.int32, (K_CAND, K_CAND), 0)
                    <= lax.broadcasted_iota(jnp.int32, (K_CAND, K_CAND), 1),
                    1.0, 0.0)
    kc = jnp.dot(keepf, tri, preferred_element_type=jnp.float32)
    nc = jnp.dot(1.0 - keepf, tri, preferred_element_type=jnp.float32)
    nk_tot = kc[:, K_CAND - 1:K_CAND]
    rank = jnp.where(keep, kc - 1.0, nk_tot + nc - 1.0)

    srange = lax.broadcasted_iota(jnp.int32, (1, K_KEEP, 1), 1
                                  ).astype(jnp.float32)
    M = jnp.where(rank[:, None, :] == srange, 1.0, 0.0)
    v = jnp.sum(M * keepf[:, None, :], axis=2)
    sx1 = jnp.sum(M * bx1[:, None, :], axis=2) * v
    sy1 = jnp.sum(M * by1[:, None, :], axis=2) * v
    sx2 = jnp.sum(M * bx2[:, None, :], axis=2) * v
    sy2 = jnp.sum(M * by2[:, None, :], axis=2) * v
    sbx1_ref[...] = sx1
    sby1_ref[...] = sy1
    sbx2_ref[...] = sx2
    sby2_ref[...] = sy2

    pio = lax.broadcasted_iota(jnp.int32, (NPROB, 1), 0)
    li_c = pio % 4
    b_c = pio // 4
    stride_c = (1 << (li_c + 2)).astype(jnp.float32)
    w_c = lax.shift_right_logical(128, li_c)[:, :, None]
    h_c = w_c
    hw_c = lax.shift_right_logical(16384, 2 * li_c)
    off_c = jnp.where(li_c == 0, 0,
                      jnp.where(li_c == 1, _LVL_OFF[1],
                                jnp.where(li_c == 2, _LVL_OFF[2], _LVL_OFF[3])))
    base_c = (off_c + b_c * hw_c)[:, :, None, None]

    x1s = sx1 / stride_c
    y1s = sy1 / stride_c
    x2s = sx2 / stride_c
    y2s = sy2 / stride_c
    bw = jnp.maximum(x2s - x1s, 1e-3)
    bh = jnp.maximum(y2s - y1s, 1e-3)
    kio = lax.broadcasted_iota(jnp.int32, (1, 1, OUT), 2
                               ).astype(jnp.float32) + 0.5
    xs = x1s[:, :, None] + kio * bw[:, :, None] / OUT - 0.5
    ys = y1s[:, :, None] + kio * bh[:, :, None] / OUT - 0.5
    x0f = jnp.floor(xs)
    y0f = jnp.floor(ys)
    wx = xs - x0f
    wy = ys - y0f
    x0i = jnp.clip(x0f.astype(jnp.int32), 0, w_c - 1)
    x1i = jnp.clip(x0i + 1, 0, w_c - 1)
    y0i = jnp.clip(y0f.astype(jnp.int32), 0, h_c - 1)
    y1i = jnp.clip(y0i + 1, 0, h_c - 1)

    y0w = y0i * w_c
    y1w = y1i * w_c
    i00_ref[...] = base_c + y0w[:, :, :, None] + x0i[:, :, None, :]
    i01_ref[...] = base_c + y0w[:, :, :, None] + x1i[:, :, None, :]
    i10_ref[...] = base_c + y1w[:, :, :, None] + x0i[:, :, None, :]
    i11_ref[...] = base_c + y1w[:, :, :, None] + x1i[:, :, None, :]
    vw = v[:, :, None, None]
    w00_ref[...] = (1.0 - wy)[:, :, :, None] * (1.0 - wx)[:, :, None, :] * vw
    w01_ref[...] = (1.0 - wy)[:, :, :, None] * wx[:, :, None, :] * vw
    w10_ref[...] = wy[:, :, :, None] * (1.0 - wx)[:, :, None, :] * vw
    w11_ref[...] = wy[:, :, :, None] * wx[:, :, None, :] * vw


def _nms_select(sorted_cols, interpret=False):
    rows = [jnp.transpose(a) for a in sorted_cols]
    sb = jax.ShapeDtypeStruct((NPROB, K_KEEP), jnp.float32)
    idx = jax.ShapeDtypeStruct((NPROB, K_KEEP, OUT, OUT), jnp.int32)
    wgt = jax.ShapeDtypeStruct((NPROB, K_KEEP, OUT, OUT), jnp.float32)
    return pl.pallas_call(
        _nms_body,
        out_shape=(sb, sb, sb, sb, idx, idx, idx, idx, wgt, wgt, wgt, wgt),
        scratch_shapes=[pltpu.VMEM((K_CAND, NPROB, K_CAND), jnp.float32),
                        pltpu.VMEM((NPROB, K_CAND), jnp.float32)],
        interpret=interpret,
    )(*rows, *sorted_cols)


_SC_TILES = 32
TOT4 = 4 * TOT_ROWS
_TILE_ROWS = TOT4 // _SC_TILES
_SC_CHUNK = 112
_SC_STEPS = _TILE_ROWS // _SC_CHUNK


def _roi_sc_body(table, idx, out, ivall, rb0, rb1, gs0, gs1, ws0, ws1):
    wid = lax.axis_index("s") * 2 + lax.axis_index("c")
    base = wid * _TILE_ROWS

    pltpu.sync_copy(idx.at[pl.ds(base, _TILE_ROWS)], ivall)

    bufs = (rb0, rb1)
    gsems = (gs0, gs1)
    wsems = (ws0, ws1)
    pending_wb = [None, None]

    g_prev = pltpu.async_copy(
        table.at[ivall.at[pl.ds(0, _SC_CHUNK)]], rb0, gs0)
    for i in range(1, _SC_STEPS + 1):
        cur = (i - 1) % 2
        nxt = i % 2
        if i < _SC_STEPS:
            if pending_wb[nxt] is not None:
                pending_wb[nxt].wait()
            g_next = pltpu.async_copy(
                table.at[ivall.at[pl.ds(i * _SC_CHUNK, _SC_CHUNK)]],
                bufs[nxt], gsems[nxt])
        g_prev.wait()
        pending_wb[cur] = pltpu.async_copy(
            bufs[cur], out.at[pl.ds(base + (i - 1) * _SC_CHUNK, _SC_CHUNK)],
            wsems[cur])
        if i < _SC_STEPS:
            g_prev = g_next
    pending_wb[0].wait()
    pending_wb[1].wait()


def _roi_sc(table, idx):
    mesh = plsc.VectorSubcoreMesh(core_axis_name="c", subcore_axis_name="s",
                                  num_cores=2, num_subcores=16)
    fn = pl.kernel(
        _roi_sc_body,
        out_type=jax.ShapeDtypeStruct((TOT4, C), jnp.float32),
        mesh=mesh,
        scratch_types=[
            pltpu.VMEM((_TILE_ROWS,), jnp.int32),
            pltpu.VMEM((_SC_CHUNK, C), jnp.float32),
            pltpu.VMEM((_SC_CHUNK, C), jnp.float32),
            pltpu.SemaphoreType.DMA,
            pltpu.SemaphoreType.DMA,
            pltpu.SemaphoreType.DMA,
            pltpu.SemaphoreType.DMA,
        ],
    )
    return fn(table, idx)


_CMB_ROWS = 1024


def _combine_body(g0_ref, g1_ref, g2_ref, g3_ref,
                  w0_ref, w1_ref, w2_ref, w3_ref, out_ref):
    out_ref[...] = (w0_ref[...] * g0_ref[...].astype(jnp.float32)
                    + w1_ref[...] * g1_ref[...].astype(jnp.float32)
                    + w2_ref[...] * g2_ref[...].astype(jnp.float32)
                    + w3_ref[...] * g3_ref[...].astype(jnp.float32))


def _combine(G, ws, interpret=False):
    nblk = TOT_ROWS // _CMB_ROWS
    g_spec = pl.BlockSpec((_CMB_ROWS, C), lambda t: (t, 0))
    gk_specs = [pl.BlockSpec((_CMB_ROWS, C),
                             functools.partial(lambda k, t: (k * nblk + t, 0),
                                               k))
                for k in range(4)]
    w_spec = pl.BlockSpec((_CMB_ROWS, 1), lambda t: (t, 0))
    return pl.pallas_call(
        _combine_body,
        grid=(nblk,),
        in_specs=gk_specs + [w_spec] * 4,
        out_specs=g_spec,
        out_shape=jax.ShapeDtypeStruct((TOT_ROWS, C), jnp.float32),
        interpret=interpret,
    )(G, G, G, G, *ws)


def _grid_layout(planes_per_level, broadcast_levels):
    flat = jnp.concatenate(planes_per_level, axis=-1)
    if broadcast_levels:
        flat = jnp.broadcast_to(flat[:, None, :], (B, 4, N_RAW))
        pad_val = 0.0
    else:
        pad_val = -jnp.inf
    flat = jnp.pad(flat, ((0, 0), (0, 0), (0, N_PAD - N_RAW)),
                   constant_values=pad_val)
    g = flat.reshape(B, 4, K_CAND, K_CAND)
    return g.transpose(2, 0, 1, 3).reshape(K_CAND, NPROB * K_CAND)


def kernel(p32, p16, p8, p4, W_obj_p4, W_reg_p4, W_obj_p8, W_reg_p8,
           W_obj_p16, W_reg_p16, W_obj_p32, W_reg_p32, _interpret=False):
    feats = (p4, p8, p16, p32)
    wobjs = (W_obj_p4, W_obj_p8, W_obj_p16, W_obj_p32)
    wregs = (W_reg_p4, W_reg_p8, W_reg_p16, W_reg_p32)
    tiles = (32, 64, 32, 16)

    msk_list, x1_list, y1_list, x2_list, y2_list = [], [], [], [], []
    for li in range(4):
        H, W = LVL_HW[li]
        wall = jnp.concatenate(
            [wobjs[li], wregs[li].reshape(4 * A, C),
             jnp.zeros((16 - A - 4 * A, C), jnp.float32)], axis=0)
        msk, x1p, y1p, x2p, y2p = _decode_level(
            feats[li], wall, STRIDES[li], H, W, tiles[li],
            interpret=_interpret)
        msk_list.append(msk.reshape(B, 4, A * H * W))
        x1_list.append(x1p.reshape(B, A * H * W))
        y1_list.append(y1p.reshape(B, A * H * W))
        x2_list.append(x2p.reshape(B, A * H * W))
        y2_list.append(y2p.reshape(B, A * H * W))

    scg = _grid_layout([m for m in msk_list], broadcast_levels=False)
    x1g = _grid_layout(x1_list, broadcast_levels=True)
    y1g = _grid_layout(y1_list, broadcast_levels=True)
    x2g = _grid_layout(x2_list, broadcast_levels=True)
    y2g = _grid_layout(y2_list, broadcast_levels=True)

    sorted_cols = _sortmerge(scg, x1g, y1g, x2g, y2g, interpret=_interpret)
    (sx1, sy1, sx2, sy2, i00, i01, i10, i11, w00, w01, w10, w11) = _nms_select(
        sorted_cols, interpret=_interpret)

    anchors = jnp.stack([sx1, sy1, sx2, sy2], axis=-1)
    anchors = anchors.reshape(B, 4 * K_KEEP, 4)

    table = jnp.concatenate(
        [jnp.transpose(f, (0, 2, 3, 1)).reshape(-1, C) for f in feats],
        axis=0)

    idx_all = jnp.concatenate(
        [i.reshape(TOT_ROWS) for i in (i00, i01, i10, i11)])
    wflat = [w.reshape(TOT_ROWS, 1) for w in (w00, w01, w10, w11)]

    if _interpret:
        G = table[idx_all]
    else:
        G = _roi_sc(table, idx_all)
    ext_rows = _combine(G, wflat, interpret=_interpret)

    exts = ext_rows.reshape(B, 4, K_KEEP, OUT, OUT, C)
    exts = exts.transpose(0, 1, 2, 5, 3, 4).reshape(
        B, 4 * K_KEEP, C, OUT, OUT)
    return exts, anchors
